# baseline (speedup 1.0000x reference)
import functools

import jax
import jax.numpy as jnp
from jax import lax
from jax.experimental import pallas as pl
from jax.experimental.pallas import tpu as pltpu
from jax.experimental.pallas import tpu_sc as plsc

_VOCAB = 100002
_HIDDEN = 32
_SAMPLES = 20
_BATCH = 16384
_NC, _NS = 2, 16
_NW = _NC * _NS
_IPW = _BATCH // _NW
_MI = 64
_NMACRO = _IPW // _MI
_PACK = 128 // _HIDDEN

_sc_mesh = plsc.VectorSubcoreMesh(core_axis_name="c", subcore_axis_name="s")


@functools.partial(
    pl.kernel,
    mesh=_sc_mesh,
    out_type=jax.ShapeDtypeStruct((_SAMPLES, _BATCH, _HIDDEN), jnp.float32),
    scratch_types=[
        pltpu.VMEM((2, _SAMPLES, _MI), jnp.int32),
        pltpu.VMEM((2, _SAMPLES, _MI, _HIDDEN), jnp.float32),
        pltpu.SemaphoreType.DMA,
        pltpu.SemaphoreType.DMA,
    ],
    compiler_params=pltpu.CompilerParams(
        use_tc_tiling_on_sc=False, needs_layout_passes=False
    ),
)
def _gather_sc(idx_hbm, table_hbm, out_hbm, idx_v, rows_v, gsem, wsem):
    wid = lax.axis_index("s") * _NC + lax.axis_index("c")
    i0 = wid * _IPW

    writebacks = []
    for m in range(_NMACRO):
        p = m % 2
        r0 = i0 + m * _MI
        pltpu.sync_copy(idx_hbm.at[:, pl.ds(r0, _MI)], idx_v.at[p])
        gathers = [
            pltpu.async_copy(
                table_hbm.at[idx_v.at[p, s]], rows_v.at[p, s], gsem
            )
            for s in range(_SAMPLES)
        ]
        if writebacks:
            writebacks[-1].wait()
        for cp in gathers:
            cp.wait()
        writebacks.append(
            pltpu.async_copy(
                rows_v.at[p], out_hbm.at[:, pl.ds(r0, _MI)], wsem
            )
        )
    writebacks[-1].wait()


def _mm_body(x_ref, bd_ref, b_ref, o_ref):
    acc = b_ref[...].astype(jnp.float32)
    for s in range(_SAMPLES):
        acc = acc + lax.dot_general(
            x_ref[s], bd_ref[s],
            (((1,), (0,)), ((), ())),
            preferred_element_type=jnp.float32,
        )
    o_ref[...] = acc


_BM4 = 512


def _project_tc(x3, BD, b128):
    return pl.pallas_call(
        _mm_body,
        grid=(_BATCH // _PACK // _BM4,),
        in_specs=[
            pl.BlockSpec((_SAMPLES, _BM4, 128), lambda i: (0, i, 0)),
            pl.BlockSpec((_SAMPLES, 128, 128), lambda i: (0, 0, 0)),
            pl.BlockSpec((1, 128), lambda i: (0, 0)),
        ],
        out_specs=pl.BlockSpec((_BM4, 128), lambda i: (i, 0)),
        out_shape=jax.ShapeDtypeStruct((_BATCH // _PACK, 128), jnp.float32),
    )(x3, BD, b128)


def kernel(all_values, table, W, b):
    idx_t = all_values.T
    emb = _gather_sc(idx_t, table)
    x3 = emb.reshape(_SAMPLES, _BATCH // _PACK, 128)
    WsT = W.reshape(_HIDDEN, _SAMPLES, _HIDDEN).transpose(1, 2, 0)
    eye4 = jnp.eye(_PACK, dtype=W.dtype)
    BD = jnp.einsum("de,sfh->sdfeh", eye4, WsT).reshape(_SAMPLES, 128, 128)
    b128 = jnp.tile(b, _PACK).reshape(1, 128)
    out = _project_tc(x3, BD, b128)
    return out.reshape(_BATCH, _HIDDEN)
